# baseline (speedup 1.0000x reference)
import functools

import jax
import jax.numpy as jnp
from jax import lax
from jax.experimental import pallas as pl
from jax.experimental.pallas import tpu as pltpu
from jax.experimental.pallas import tpu_sc as plsc

BATCH = 16384
NUM_FACTORS = 32
NUM_WORKERS = 32
PER_WORKER = BATCH // NUM_WORKERS
N_CHUNKS = 4
CHUNK = PER_WORKER // N_CHUNKS
GROUPS = PER_WORKER // 16


def _sc_body(uid_hbm, iid_hbm, exp_hbm, uemb_hbm, iemb_hbm, gamma_hbm,
             bias_hbm, out_hbm,
             uid_v, iid_v, exp_v, urows_v, irows_v, gam_v, bias_v, out_v,
             sem, sem2):
    n_cores = 2
    wid = lax.axis_index("s") * n_cores + lax.axis_index("c")
    base = wid * PER_WORKER

    pltpu.sync_copy(uid_hbm.at[wid], uid_v)
    pltpu.sync_copy(iid_hbm.at[wid], iid_v)
    pltpu.sync_copy(exp_hbm.at[pl.ds(base, PER_WORKER)], exp_v)
    pltpu.sync_copy(bias_hbm, bias_v)

    copies = []
    for j in range(N_CHUNKS):
        copies.append(pltpu.async_copy(
            uemb_hbm.at[uid_v.at[j]],
            urows_v.at[pl.ds(j * CHUNK, CHUNK)], sem))
        copies.append(pltpu.async_copy(
            iemb_hbm.at[iid_v.at[j]],
            irows_v.at[pl.ds(j * CHUNK, CHUNK)], sem))
        copies.append(pltpu.async_copy(
            gamma_hbm.at[uid_v.at[j]],
            gam_v.at[pl.ds(j * CHUNK, CHUNK)], sem2))
    for c in copies:
        c.wait()

    lane = lax.iota(jnp.int32, 16)
    bias_vec = bias_v[...]

    def group(g, _):
        row = g * 16 + lane
        acc = gam_v[pl.ds(g * 16, 16)] * exp_v[pl.ds(g * 16, 16)] + bias_vec
        for d in range(NUM_FACTORS):
            col = jnp.full((16,), d, jnp.int32)
            u = plsc.load_gather(urows_v, [row, col])
            v = plsc.load_gather(irows_v, [row, col])
            acc = acc + u * v
        out_v[pl.ds(g * 16, 16)] = acc
        return _

    lax.fori_loop(0, GROUPS, group, None)

    pltpu.sync_copy(out_v, out_hbm.at[pl.ds(base, PER_WORKER)])


@jax.jit
def kernel(user_ids, item_ids, exposures_hat, user_embeddings,
           item_embeddings, gamma, bias):
    mesh = plsc.VectorSubcoreMesh(core_axis_name="c", subcore_axis_name="s")
    uid3 = user_ids.reshape(NUM_WORKERS, N_CHUNKS, CHUNK)
    iid3 = item_ids.reshape(NUM_WORKERS, N_CHUNKS, CHUNK)
    bias16 = jnp.broadcast_to(bias, (16,))
    run = functools.partial(
        pl.kernel,
        mesh=mesh,
        compiler_params=pltpu.CompilerParams(
            needs_layout_passes=False, use_tc_tiling_on_sc=False),
        out_type=jax.ShapeDtypeStruct((BATCH,), jnp.float32),
        scratch_types=[
            pltpu.VMEM((N_CHUNKS, CHUNK), jnp.int32),
            pltpu.VMEM((N_CHUNKS, CHUNK), jnp.int32),
            pltpu.VMEM((PER_WORKER,), jnp.float32),
            pltpu.VMEM((PER_WORKER, NUM_FACTORS), jnp.float32),
            pltpu.VMEM((PER_WORKER, NUM_FACTORS), jnp.float32),
            pltpu.VMEM((PER_WORKER,), jnp.float32),
            pltpu.VMEM((16,), jnp.float32),
            pltpu.VMEM((PER_WORKER,), jnp.float32),
            pltpu.SemaphoreType.DMA,
            pltpu.SemaphoreType.DMA,
        ],
    )(_sc_body)
    return run(uid3, iid3, exposures_hat, user_embeddings, item_embeddings,
               gamma, bias16)
